# baseline (speedup 1.0000x reference)
import functools

import jax
import jax.numpy as jnp
from jax import lax
from jax.experimental import pallas as pl
from jax.experimental.pallas import tpu as pltpu
from jax.experimental.pallas import tpu_sc as plsc

B = 16384
D = 128
NC = 2
NS = 16
L = 16
NW = NC * NS
BPW = B // NW
CH = 64
NCH = BPW // CH
NSLOT = 4

_MESH = plsc.VectorSubcoreMesh(core_axis_name="c", subcore_axis_name="s")


def _bpr_body(user_h, pos_h, neg_h, ue_h, ie_h, pos_o, neg_o,
              uidx, pidx, nidx, bufs, pout, nout, sems):
    wid = lax.axis_index("s") * NC + lax.axis_index("c")
    base = wid * BPW

    pltpu.sync_copy(user_h.at[pl.ds(base, BPW)], uidx)
    pltpu.sync_copy(pos_h.at[pl.ds(base, BPW)], pidx)
    pltpu.sync_copy(neg_h.at[pl.ds(base, BPW)], nidx)

    def fetch(j, slot):
        ub, pb, nb = bufs[slot]
        c1 = pltpu.async_copy(ue_h.at[uidx.at[pl.ds(j * CH, CH)]], ub, sems[slot])
        c2 = pltpu.async_copy(ie_h.at[pidx.at[pl.ds(j * CH, CH)]], pb, sems[slot])
        c3 = pltpu.async_copy(ie_h.at[nidx.at[pl.ds(j * CH, CH)]], nb, sems[slot])
        return (c1, c2, c3)

    lane15 = lax.iota(jnp.int32, L) == (L - 1)

    pending = [fetch(j, j) for j in range(NSLOT - 1)]
    for j in range(NCH):
        slot = j % NSLOT
        current = pending.pop(0)
        if j + NSLOT - 1 < NCH:
            pending.append(fetch(j + NSLOT - 1, (j + NSLOT - 1) % NSLOT))
        for c in current:
            c.wait()
        ub, pb, nb = bufs[slot]

        def row(r, carry, ub=ub, pb=pb, nb=nb, j=j):
            us = [ub[r, pl.ds(cc * L, L)] for cc in range(D // L)]
            ps = [pb[r, pl.ds(cc * L, L)] for cc in range(D // L)]
            ns = [nb[r, pl.ds(cc * L, L)] for cc in range(D // L)]
            pprod = [us[cc] * ps[cc] for cc in range(D // L)]
            nprod = [us[cc] * ns[cc] for cc in range(D // L)]
            while len(pprod) > 1:
                pprod = [a + b for a, b in zip(pprod[0::2], pprod[1::2])]
                nprod = [a + b for a, b in zip(nprod[0::2], nprod[1::2])]
            out_idx = jnp.broadcast_to(j * CH + r, (L,)).astype(jnp.int32)
            plsc.store_scatter(pout, [out_idx], plsc.cumsum(pprod[0]), mask=lane15)
            plsc.store_scatter(nout, [out_idx], plsc.cumsum(nprod[0]), mask=lane15)
            return carry

        lax.fori_loop(0, CH, row, 0)

    pltpu.sync_copy(pout, pos_o.at[pl.ds(base, BPW)])
    pltpu.sync_copy(nout, neg_o.at[pl.ds(base, BPW)])


def _body_wrapper(user_h, pos_h, neg_h, ue_h, ie_h, pos_o, neg_o,
                  uidx, pidx, nidx,
                  u0, p0, n0, u1, p1, n1, u2, p2, n2, u3, p3, n3,
                  pout, nout, s0, s1, s2, s3):
    bufs = ((u0, p0, n0), (u1, p1, n1), (u2, p2, n2), (u3, p3, n3))
    _bpr_body(user_h, pos_h, neg_h, ue_h, ie_h, pos_o, neg_o,
              uidx, pidx, nidx, bufs, pout, nout, (s0, s1, s2, s3))


_bpr = pl.kernel(
    _body_wrapper,
    out_type=[
        jax.ShapeDtypeStruct((B,), jnp.float32),
        jax.ShapeDtypeStruct((B,), jnp.float32),
    ],
    mesh=_MESH,
    compiler_params=pltpu.CompilerParams(needs_layout_passes=False),
    scratch_types=(
        [pltpu.VMEM((BPW,), jnp.int32)] * 3
        + [pltpu.VMEM((CH, D), jnp.float32)] * (3 * NSLOT)
        + [pltpu.VMEM((BPW,), jnp.float32)] * 2
        + [pltpu.SemaphoreType.DMA] * NSLOT
    ),
)


@jax.jit
def kernel(user, pos_item, neg_item, user_embedding, item_embedding):
    user = user.astype(jnp.int32)
    pos_item = pos_item.astype(jnp.int32)
    neg_item = neg_item.astype(jnp.int32)
    pos_pred, neg_pred = _bpr(user, pos_item, neg_item,
                              user_embedding, item_embedding)
    return (pos_pred, neg_pred)


# speedup vs baseline: 1.1923x; 1.1923x over previous
import jax
import jax.numpy as jnp
from jax import lax
from jax.experimental import pallas as pl
from jax.experimental.pallas import tpu as pltpu
from jax.experimental.pallas import tpu_sc as plsc

B = 16384
D = 128
NC = 2
NS = 16
L = 16
NW = NC * NS
BPW = B // NW
CH = 64
NCH = BPW // CH
NSLOT = 2
NG = NCH // NSLOT

_MESH = plsc.VectorSubcoreMesh(core_axis_name="c", subcore_axis_name="s")


def _bpr_body(user_h, pos_h, neg_h, ue_h, ie_h, pos_o, neg_o,
              uidx, pidx, nidx,
              u0, p0, n0, u1, p1, n1,
              pout, nout, s0, s1):
    wid = lax.axis_index("s") * NC + lax.axis_index("c")
    base = wid * BPW
    bufs = ((u0, p0, n0), (u1, p1, n1))
    sems = (s0, s1)

    pltpu.sync_copy(user_h.at[pl.ds(base, BPW)], uidx)
    pltpu.sync_copy(pos_h.at[pl.ds(base, BPW)], pidx)
    pltpu.sync_copy(neg_h.at[pl.ds(base, BPW)], nidx)

    def fetch(j, slot):
        off = pl.multiple_of(j * CH, CH)
        ub, pb, nb = bufs[slot]
        pltpu.async_copy(ue_h.at[uidx.at[pl.ds(off, CH)]], ub, sems[slot])
        pltpu.async_copy(ie_h.at[pidx.at[pl.ds(off, CH)]], pb, sems[slot])
        pltpu.async_copy(ie_h.at[nidx.at[pl.ds(off, CH)]], nb, sems[slot])

    def drain(slot):
        ub, pb, nb = bufs[slot]
        pltpu.make_async_copy(ue_h.at[uidx.at[pl.ds(0, CH)]], ub, sems[slot]).wait()
        pltpu.make_async_copy(ie_h.at[pidx.at[pl.ds(0, CH)]], pb, sems[slot]).wait()
        pltpu.make_async_copy(ie_h.at[nidx.at[pl.ds(0, CH)]], nb, sems[slot]).wait()

    lane15 = lax.iota(jnp.int32, L) == (L - 1)

    for slot in range(NSLOT):
        fetch(slot, slot)

    @pl.loop(0, NG)
    def ring(g):
        for slot in range(NSLOT):
            j = g * NSLOT + slot
            drain(slot)
            ub, pb, nb = bufs[slot]
            obase = j * CH

            @plsc.parallel_loop(0, CH, unroll=2)
            def row(r, ub=ub, pb=pb, nb=nb, obase=obase):
                us = [ub[r, pl.ds(cc * L, L)] for cc in range(D // L)]
                ps = [pb[r, pl.ds(cc * L, L)] for cc in range(D // L)]
                ns = [nb[r, pl.ds(cc * L, L)] for cc in range(D // L)]
                pprod = [us[cc] * ps[cc] for cc in range(D // L)]
                nprod = [us[cc] * ns[cc] for cc in range(D // L)]
                while len(pprod) > 1:
                    pprod = [a + b for a, b in zip(pprod[0::2], pprod[1::2])]
                    nprod = [a + b for a, b in zip(nprod[0::2], nprod[1::2])]
                out_idx = jnp.broadcast_to(obase + r, (L,)).astype(jnp.int32)
                plsc.store_scatter(pout, [out_idx], plsc.cumsum(pprod[0]), mask=lane15)
                plsc.store_scatter(nout, [out_idx], plsc.cumsum(nprod[0]), mask=lane15)

            @pl.when(g < NG - 1)
            def _prefetch(j=j, slot=slot):
                fetch(j + NSLOT, slot)

    pltpu.sync_copy(pout, pos_o.at[pl.ds(base, BPW)])
    pltpu.sync_copy(nout, neg_o.at[pl.ds(base, BPW)])


_bpr = pl.kernel(
    _bpr_body,
    out_type=[
        jax.ShapeDtypeStruct((B,), jnp.float32),
        jax.ShapeDtypeStruct((B,), jnp.float32),
    ],
    mesh=_MESH,
    compiler_params=pltpu.CompilerParams(needs_layout_passes=False),
    scratch_types=(
        [pltpu.VMEM((BPW,), jnp.int32)] * 3
        + [pltpu.VMEM((CH, D), jnp.float32)] * (3 * NSLOT)
        + [pltpu.VMEM((BPW,), jnp.float32)] * 2
        + [pltpu.SemaphoreType.DMA] * NSLOT
    ),
)


@jax.jit
def kernel(user, pos_item, neg_item, user_embedding, item_embedding):
    user = user.astype(jnp.int32)
    pos_item = pos_item.astype(jnp.int32)
    neg_item = neg_item.astype(jnp.int32)
    pos_pred, neg_pred = _bpr(user, pos_item, neg_item,
                              user_embedding, item_embedding)
    return (pos_pred, neg_pred)
